# baseline (speedup 1.0000x reference)
import functools

import jax
import jax.numpy as jnp
from jax import lax
from jax.experimental import pallas as pl
from jax.experimental.pallas import tpu as pltpu

_TILE_V = 2048
_N_CHUNKS = 8


def _body(tok_ref, table_ref, w_ref, b_ref, o_ref, x_ref, sems, *, m, n_chunks):
    j = pl.program_id(0)
    c = pl.program_id(1)

    def issue_chunk(cc):
        def issue_one(i, carry):
            row = cc * m + i
            t = tok_ref[row]
            pltpu.make_async_copy(
                table_ref.at[pl.ds(t, 1), :],
                x_ref.at[pl.ds(row, 1), :],
                sems.at[cc],
            ).start()
            return carry

        lax.fori_loop(0, m, issue_one, 0)

    @pl.when((j == 0) & (c == 0))
    def _():
        issue_chunk(0)
        issue_chunk(1)

    @pl.when((j == 0) & (c > 0) & (c < n_chunks - 1))
    def _():
        issue_chunk(c + 1)

    @pl.when(j == 0)
    def _():
        pltpu.make_async_copy(
            table_ref.at[pl.ds(0, m), :],
            x_ref.at[pl.ds(c * m, m), :],
            sems.at[c],
        ).wait()

    x = x_ref[pl.ds(c * m, m), :]
    o_ref[...] = (
        lax.dot_general(
            x, w_ref[...], (((1,), (1,)), ((), ())),
            preferred_element_type=jnp.float32,
        )
        + b_ref[...]
    )


def kernel(input_token, emb_table, fc_weight, fc_bias):
    V, D = emb_table.shape
    B = input_token.shape[0]
    tokens = input_token.astype(jnp.int32)
    n_chunks = _N_CHUNKS
    m = B // n_chunks
    grid_j = pl.cdiv(V, _TILE_V)

    grid_spec = pltpu.PrefetchScalarGridSpec(
        num_scalar_prefetch=1,
        grid=(grid_j, n_chunks),
        in_specs=[
            pl.BlockSpec(memory_space=pl.ANY),
            pl.BlockSpec((_TILE_V, D), lambda j, c, tok: (j, 0)),
            pl.BlockSpec((1, _TILE_V), lambda j, c, tok: (0, j)),
        ],
        out_specs=pl.BlockSpec((m, _TILE_V), lambda j, c, tok: (c, j)),
        scratch_shapes=[
            pltpu.VMEM((B, D), jnp.float32),
            pltpu.SemaphoreType.DMA((n_chunks,)),
        ],
    )
    return pl.pallas_call(
        functools.partial(_body, m=m, n_chunks=n_chunks),
        grid_spec=grid_spec,
        out_shape=jax.ShapeDtypeStruct((B, V), jnp.float32),
        compiler_params=pltpu.CompilerParams(
            dimension_semantics=("arbitrary", "arbitrary"),
        ),
    )(tokens, emb_table, fc_weight, fc_bias.reshape(1, V))


# speedup vs baseline: 3.8160x; 3.8160x over previous
import functools

import jax
import jax.numpy as jnp
from jax import lax
from jax.experimental import pallas as pl
from jax.experimental.pallas import tpu as pltpu

_TILE_V = 2048


def _body(stok_ref, order_ref, starts_ref, tbl_ref, w_ref, b_ref, o_ref,
          x_ref, tpose_ref, sem, *, tile_v):
    p = pl.program_id(0)
    j = pl.program_id(1)

    @pl.when(p == 0)
    def _gather():
        tpose_ref[...] = jnp.transpose(tbl_ref[...])
        lo = j * tile_v

        def copy_one(i, carry):
            local = stok_ref[i] - lo
            dst = order_ref[i]
            d = pltpu.make_async_copy(
                tpose_ref.at[pl.ds(local, 1), :],
                x_ref.at[pl.ds(dst, 1), :],
                sem,
            )
            d.start()
            d.wait()
            return carry

        lax.fori_loop(starts_ref[j], starts_ref[j + 1], copy_one, 0)

    @pl.when(p == 1)
    def _matmul():
        acc = lax.dot_general(
            w_ref[...],
            x_ref[...],
            (((0,), (1,)), ((), ())),
            preferred_element_type=jnp.float32,
        )
        o_ref[...] = acc + jnp.transpose(b_ref[...])


def kernel(input_token, emb_table, fc_weight, fc_bias):
    V, D = emb_table.shape
    B = input_token.shape[0]
    tile_v = _TILE_V
    grid_j = pl.cdiv(V, tile_v)

    tokens = input_token.astype(jnp.int32)
    order = jnp.argsort(tokens).astype(jnp.int32)
    sorted_tok = jnp.take(tokens, order)
    starts = jnp.searchsorted(
        sorted_tok, jnp.arange(grid_j + 1, dtype=jnp.int32) * tile_v
    ).astype(jnp.int32)

    table_t = emb_table.T
    w_t = fc_weight.T
    bias2d = fc_bias.reshape(1, V)

    grid_spec = pltpu.PrefetchScalarGridSpec(
        num_scalar_prefetch=3,
        grid=(2, grid_j),
        in_specs=[
            pl.BlockSpec(
                (D, tile_v), lambda p, j, *_: (0, jnp.where(p == 0, j, 0))
            ),
            pl.BlockSpec(
                (D, tile_v), lambda p, j, *_: (0, jnp.where(p == 1, j, 0))
            ),
            pl.BlockSpec(
                (1, tile_v), lambda p, j, *_: (0, jnp.where(p == 1, j, 0))
            ),
        ],
        out_specs=pl.BlockSpec(
            (tile_v, B), lambda p, j, *_: (jnp.where(p == 1, j, 0), 0)
        ),
        scratch_shapes=[
            pltpu.VMEM((B, D), jnp.float32),
            pltpu.VMEM((tile_v, D), jnp.float32),
            pltpu.SemaphoreType.DMA,
        ],
    )
    out_t = pl.pallas_call(
        functools.partial(_body, tile_v=tile_v),
        grid_spec=grid_spec,
        out_shape=jax.ShapeDtypeStruct((V, B), jnp.float32),
        compiler_params=pltpu.CompilerParams(
            dimension_semantics=("arbitrary", "arbitrary"),
        ),
    )(sorted_tok, order, starts, table_t, w_t, bias2d)
    return out_t.T
